# baseline (speedup 1.0000x reference)
import functools

import jax
import jax.numpy as jnp
from jax import lax
from jax.experimental import pallas as pl
from jax.experimental.pallas import tpu as pltpu
from jax.experimental.pallas import tpu_sc as plsc

D_EMB = 128
NC, NS = 2, 16
NW = NC * NS
CHUNK = 128
K_PIPE = 5


def _gather_body(table_hbm, idx_hbm, out_hbm, idx_v, rows0, rows1, sem0, sem1):
    wid = lax.axis_index("s") * NC + lax.axis_index("c")
    n_chunks = idx_v.shape[0] // CHUNK
    base = wid * (n_chunks * CHUNK)
    pltpu.sync_copy(idx_hbm.at[wid], idx_v)

    def idx_slice(c):
        return idx_v.at[pl.ds(c * CHUNK, CHUNK)]

    pltpu.async_copy(table_hbm.at[idx_slice(0)], rows0, sem0)
    pltpu.async_copy(table_hbm.at[idx_slice(1)], rows1, sem1)

    def body(i, carry):
        c0 = 2 * i
        pltpu.make_async_copy(table_hbm.at[idx_slice(c0)], rows0, sem0).wait()
        pltpu.sync_copy(rows0, out_hbm.at[pl.ds(base + c0 * CHUNK, CHUNK)])

        @pl.when(c0 + 2 < n_chunks)
        def _():
            pltpu.async_copy(table_hbm.at[idx_slice(c0 + 2)], rows0, sem0)

        pltpu.make_async_copy(table_hbm.at[idx_slice(c0 + 1)], rows1, sem1).wait()
        pltpu.sync_copy(rows1, out_hbm.at[pl.ds(base + (c0 + 1) * CHUNK, CHUNK)])

        @pl.when(c0 + 3 < n_chunks)
        def _():
            pltpu.async_copy(table_hbm.at[idx_slice(c0 + 3)], rows1, sem1)

        return carry

    lax.fori_loop(0, n_chunks // 2, body, 0)


def _make_gather(n_rows):
    rows_per_w = n_rows // NW
    mesh = plsc.VectorSubcoreMesh(core_axis_name="c", subcore_axis_name="s")
    return pl.kernel(
        _gather_body,
        out_type=jax.ShapeDtypeStruct((n_rows, D_EMB), jnp.float32),
        scratch_types=[
            pltpu.VMEM((rows_per_w,), jnp.int32),
            pltpu.VMEM((CHUNK, D_EMB), jnp.float32),
            pltpu.VMEM((CHUNK, D_EMB), jnp.float32),
            pltpu.SemaphoreType.DMA,
            pltpu.SemaphoreType.DMA,
        ],
        mesh=mesh,
    )


def _mlp_body(h_ref, w1_ref, b1_ref, w2_ref, b2_ref, out_ref):
    h = h_ref[...]
    z = lax.dot_general(h, w1_ref[...], (((1,), (1,)), ((), ())),
                        preferred_element_type=jnp.float32)
    z = jnp.maximum(z + b1_ref[...], 0.0)
    o = lax.dot_general(z, w2_ref[...], (((1,), (1,)), ((), ())),
                        preferred_element_type=jnp.float32)
    out_ref[...] = o + b2_ref[...]


def _mlp(h, W1, b1, W2, b2, blk=2048):
    n_rows = h.shape[0]
    n_tags = W2.shape[0]
    grid = (n_rows // blk,)
    return pl.pallas_call(
        _mlp_body,
        grid=grid,
        in_specs=[
            pl.BlockSpec((blk, D_EMB), lambda i: (i, 0)),
            pl.BlockSpec((D_EMB, D_EMB), lambda i: (0, 0)),
            pl.BlockSpec((1, D_EMB), lambda i: (0, 0)),
            pl.BlockSpec((n_tags, D_EMB), lambda i: (0, 0)),
            pl.BlockSpec((1, n_tags), lambda i: (0, 0)),
        ],
        out_specs=pl.BlockSpec((blk, n_tags), lambda i: (i, 0)),
        out_shape=jax.ShapeDtypeStruct((n_rows, n_tags), jnp.float32),
    )(h, W1, b1, W2, b2)


def kernel(x, emb, W1, b1, W2, b2):
    B, L = x.shape
    b1r, b2r = b1.reshape(1, -1), b2.reshape(1, -1)
    x_flat = x.reshape(-1)
    nk = (B * L) // K_PIPE
    gather = _make_gather(nk)
    outs = []
    for k in range(K_PIPE):
        xk = lax.slice_in_dim(x_flat, k * nk, (k + 1) * nk, axis=0)
        idx = xk.reshape(NW, nk // NW).astype(jnp.int32)
        h = gather(emb, idx)
        outs.append(_mlp(h, W1, b1r, W2, b2r, blk=4096))
    return jnp.concatenate(outs, axis=0).reshape(B, L, -1)


# speedup vs baseline: 1.6275x; 1.6275x over previous
import functools

import jax
import jax.numpy as jnp
from jax import lax
from jax.experimental import pallas as pl
from jax.experimental.pallas import tpu as pltpu
from jax.experimental.pallas import tpu_sc as plsc

D_EMB = 128
NC, NS = 2, 16
NW = NC * NS
CHUNK = 80
K_PIPE = 4


def _gather_body(table_hbm, idx_hbm, out_hbm, idx_v, rows0, rows1, sem0, sem1):
    wid = lax.axis_index("s") * NC + lax.axis_index("c")
    n_chunks = idx_v.shape[0] // CHUNK
    base = wid * (n_chunks * CHUNK)
    pltpu.sync_copy(idx_hbm.at[wid], idx_v)

    def idx_slice(c):
        return idx_v.at[pl.ds(c * CHUNK, CHUNK)]

    pltpu.async_copy(table_hbm.at[idx_slice(0)], rows0, sem0)
    pltpu.async_copy(table_hbm.at[idx_slice(1)], rows1, sem1)

    def body(i, carry):
        c0 = 2 * i
        pltpu.make_async_copy(table_hbm.at[idx_slice(c0)], rows0, sem0).wait()
        pltpu.sync_copy(rows0, out_hbm.at[pl.ds(base + c0 * CHUNK, CHUNK)])

        @pl.when(c0 + 2 < n_chunks)
        def _():
            pltpu.async_copy(table_hbm.at[idx_slice(c0 + 2)], rows0, sem0)

        pltpu.make_async_copy(table_hbm.at[idx_slice(c0 + 1)], rows1, sem1).wait()
        pltpu.sync_copy(rows1, out_hbm.at[pl.ds(base + (c0 + 1) * CHUNK, CHUNK)])

        @pl.when(c0 + 3 < n_chunks)
        def _():
            pltpu.async_copy(table_hbm.at[idx_slice(c0 + 3)], rows1, sem1)

        return carry

    lax.fori_loop(0, n_chunks // 2, body, 0)


def _make_gather(n_rows):
    rows_per_w = n_rows // NW
    mesh = plsc.VectorSubcoreMesh(core_axis_name="c", subcore_axis_name="s")
    return pl.kernel(
        _gather_body,
        out_type=jax.ShapeDtypeStruct((n_rows, D_EMB), jnp.float32),
        scratch_types=[
            pltpu.VMEM((rows_per_w,), jnp.int32),
            pltpu.VMEM((CHUNK, D_EMB), jnp.float32),
            pltpu.VMEM((CHUNK, D_EMB), jnp.float32),
            pltpu.SemaphoreType.DMA,
            pltpu.SemaphoreType.DMA,
        ],
        mesh=mesh,
    )


def _mlp_body(h_ref, w1_ref, b1_ref, w2_ref, b2_ref, out_ref):
    h = h_ref[...]
    z = lax.dot_general(h, w1_ref[...], (((1,), (1,)), ((), ())),
                        preferred_element_type=jnp.float32)
    z = jnp.maximum(z + b1_ref[...], 0.0)
    o = lax.dot_general(z, w2_ref[...], (((1,), (1,)), ((), ())),
                        preferred_element_type=jnp.float32)
    out_ref[...] = o + b2_ref[...]


def _mlp(h, W1, b1, W2, b2, blk=2048):
    n_rows = h.shape[0]
    n_tags = W2.shape[0]
    grid = (n_rows // blk,)
    return pl.pallas_call(
        _mlp_body,
        grid=grid,
        in_specs=[
            pl.BlockSpec((blk, D_EMB), lambda i: (i, 0)),
            pl.BlockSpec((D_EMB, D_EMB), lambda i: (0, 0)),
            pl.BlockSpec((1, D_EMB), lambda i: (0, 0)),
            pl.BlockSpec((n_tags, D_EMB), lambda i: (0, 0)),
            pl.BlockSpec((1, n_tags), lambda i: (0, 0)),
        ],
        out_specs=pl.BlockSpec((blk, n_tags), lambda i: (i, 0)),
        out_shape=jax.ShapeDtypeStruct((n_rows, n_tags), jnp.float32),
    )(h, W1, b1, W2, b2)


def kernel(x, emb, W1, b1, W2, b2):
    B, L = x.shape
    b1r, b2r = b1.reshape(1, -1), b2.reshape(1, -1)
    bk = B // K_PIPE
    nk = bk * L
    gather = _make_gather(nk)
    outs = []
    for k in range(K_PIPE):
        xk = lax.slice_in_dim(x, k * bk, (k + 1) * bk, axis=0)
        idx = xk.reshape(NW, nk // NW).astype(jnp.int32)
        h = gather(emb, idx)
        o = _mlp(h, W1, b1r, W2, b2r, blk=6400)
        outs.append(o.reshape(bk, L, -1))
    return jnp.concatenate(outs, axis=0)


# speedup vs baseline: 1.7616x; 1.0825x over previous
import functools

import jax
import jax.numpy as jnp
from jax import lax
from jax.experimental import pallas as pl
from jax.experimental.pallas import tpu as pltpu
from jax.experimental.pallas import tpu_sc as plsc

D_EMB = 128
NC, NS = 2, 16
NW = NC * NS
CHUNK = 128
K_PIPE = 2


def _gather_body(table_hbm, idx_hbm, out_hbm, idx_v, rows0, rows1, sem0, sem1):
    wid = lax.axis_index("s") * NC + lax.axis_index("c")
    n_chunks = idx_v.shape[0] // CHUNK
    base = wid * (n_chunks * CHUNK)
    pltpu.sync_copy(idx_hbm.at[wid], idx_v)

    def idx_slice(c):
        return idx_v.at[pl.ds(c * CHUNK, CHUNK)]

    pltpu.async_copy(table_hbm.at[idx_slice(0)], rows0, sem0)
    pltpu.async_copy(table_hbm.at[idx_slice(1)], rows1, sem1)

    def body(i, carry):
        c0 = 2 * i
        pltpu.make_async_copy(table_hbm.at[idx_slice(c0)], rows0, sem0).wait()
        pltpu.sync_copy(rows0, out_hbm.at[pl.ds(base + c0 * CHUNK, CHUNK)])

        @pl.when(c0 + 2 < n_chunks)
        def _():
            pltpu.async_copy(table_hbm.at[idx_slice(c0 + 2)], rows0, sem0)

        pltpu.make_async_copy(table_hbm.at[idx_slice(c0 + 1)], rows1, sem1).wait()
        pltpu.sync_copy(rows1, out_hbm.at[pl.ds(base + (c0 + 1) * CHUNK, CHUNK)])

        @pl.when(c0 + 3 < n_chunks)
        def _():
            pltpu.async_copy(table_hbm.at[idx_slice(c0 + 3)], rows1, sem1)

        return carry

    lax.fori_loop(0, n_chunks // 2, body, 0)

    if n_chunks % 2 == 1:
        c = n_chunks - 1
        pltpu.make_async_copy(table_hbm.at[idx_slice(c)], rows0, sem0).wait()
        pltpu.sync_copy(rows0, out_hbm.at[pl.ds(base + c * CHUNK, CHUNK)])


def _make_gather(n_rows):
    rows_per_w = n_rows // NW
    mesh = plsc.VectorSubcoreMesh(core_axis_name="c", subcore_axis_name="s")
    return pl.kernel(
        _gather_body,
        out_type=jax.ShapeDtypeStruct((n_rows, D_EMB), jnp.float32),
        scratch_types=[
            pltpu.VMEM((rows_per_w,), jnp.int32),
            pltpu.VMEM((CHUNK, D_EMB), jnp.float32),
            pltpu.VMEM((CHUNK, D_EMB), jnp.float32),
            pltpu.SemaphoreType.DMA,
            pltpu.SemaphoreType.DMA,
        ],
        mesh=mesh,
    )


def _mlp_body(h_ref, w1_ref, b1_ref, w2_ref, b2_ref, out_ref):
    h = h_ref[...]
    z = lax.dot_general(h, w1_ref[...], (((1,), (1,)), ((), ())),
                        preferred_element_type=jnp.float32)
    z = jnp.maximum(z + b1_ref[...], 0.0)
    o = lax.dot_general(z, w2_ref[...], (((1,), (1,)), ((), ())),
                        preferred_element_type=jnp.float32)
    out_ref[...] = o + b2_ref[...]


def _mlp(h, W1, b1, W2, b2, blk=2048):
    n_rows = h.shape[0]
    n_tags = W2.shape[0]
    grid = (n_rows // blk,)
    return pl.pallas_call(
        _mlp_body,
        grid=grid,
        in_specs=[
            pl.BlockSpec((blk, D_EMB), lambda i: (i, 0)),
            pl.BlockSpec((D_EMB, D_EMB), lambda i: (0, 0)),
            pl.BlockSpec((1, D_EMB), lambda i: (0, 0)),
            pl.BlockSpec((n_tags, D_EMB), lambda i: (0, 0)),
            pl.BlockSpec((1, n_tags), lambda i: (0, 0)),
        ],
        out_specs=pl.BlockSpec((blk, n_tags), lambda i: (i, 0)),
        out_shape=jax.ShapeDtypeStruct((n_rows, n_tags), jnp.float32),
    )(h, W1, b1, W2, b2)


def kernel(x, emb, W1, b1, W2, b2):
    B, L = x.shape
    b1r, b2r = b1.reshape(1, -1), b2.reshape(1, -1)
    bk = B // K_PIPE
    nk = bk * L
    gather = _make_gather(nk)
    outs = []
    for k in range(K_PIPE):
        xk = lax.slice_in_dim(x, k * bk, (k + 1) * bk, axis=0)
        idx = xk.reshape(NW, nk // NW).astype(jnp.int32)
        h = gather(emb, idx)
        o = _mlp(h, W1, b1r, W2, b2r, blk=4096)
        outs.append(o.reshape(bk, L, -1))
    return jnp.concatenate(outs, axis=0)


# speedup vs baseline: 2.6483x; 1.5033x over previous
import functools

import jax
import jax.numpy as jnp
from jax import lax
from jax.experimental import pallas as pl
from jax.experimental.pallas import tpu as pltpu
from jax.experimental.pallas import tpu_sc as plsc

D_EMB = 128
NC, NS = 2, 16
NW = NC * NS
CHUNK = 128
K_PIPE = 2


def _gather_body(table_hbm, idx_hbm, out_hbm, idx_v, rows0, rows1, sem0, sem1):
    wid = lax.axis_index("s") * NC + lax.axis_index("c")
    n_chunks = idx_v.shape[0] // CHUNK
    base = wid * (n_chunks * CHUNK)
    pltpu.sync_copy(idx_hbm.at[wid], idx_v)

    def idx_slice(c):
        return idx_v.at[pl.ds(c * CHUNK, CHUNK)]

    pltpu.async_copy(table_hbm.at[idx_slice(0)], rows0, sem0)
    pltpu.async_copy(table_hbm.at[idx_slice(1)], rows1, sem1)

    def body(i, carry):
        c0 = 2 * i
        pltpu.make_async_copy(table_hbm.at[idx_slice(c0)], rows0, sem0).wait()
        pltpu.sync_copy(rows0, out_hbm.at[pl.ds(base + c0 * CHUNK, CHUNK)])

        @pl.when(c0 + 2 < n_chunks)
        def _():
            pltpu.async_copy(table_hbm.at[idx_slice(c0 + 2)], rows0, sem0)

        pltpu.make_async_copy(table_hbm.at[idx_slice(c0 + 1)], rows1, sem1).wait()
        pltpu.sync_copy(rows1, out_hbm.at[pl.ds(base + (c0 + 1) * CHUNK, CHUNK)])

        @pl.when(c0 + 3 < n_chunks)
        def _():
            pltpu.async_copy(table_hbm.at[idx_slice(c0 + 3)], rows1, sem1)

        return carry

    lax.fori_loop(0, n_chunks // 2, body, 0)

    if n_chunks % 2 == 1:
        c = n_chunks - 1
        pltpu.make_async_copy(table_hbm.at[idx_slice(c)], rows0, sem0).wait()
        pltpu.sync_copy(rows0, out_hbm.at[pl.ds(base + c * CHUNK, CHUNK)])


def _make_gather(n_rows):
    rows_per_w = n_rows // NW
    mesh = plsc.VectorSubcoreMesh(core_axis_name="c", subcore_axis_name="s")
    return pl.kernel(
        _gather_body,
        out_type=jax.ShapeDtypeStruct((n_rows, D_EMB), jnp.float32),
        scratch_types=[
            pltpu.VMEM((rows_per_w,), jnp.int32),
            pltpu.VMEM((CHUNK, D_EMB), jnp.float32),
            pltpu.VMEM((CHUNK, D_EMB), jnp.float32),
            pltpu.SemaphoreType.DMA,
            pltpu.SemaphoreType.DMA,
        ],
        mesh=mesh,
    )


def _mlp_body(lb, B, h_ref, w1_ref, b1_ref, w2_ref, b2_ref, out_ref):
    h = h_ref[...]
    z = lax.dot_general(h, w1_ref[...], (((1,), (1,)), ((), ())),
                        preferred_element_type=jnp.float32)
    z = jnp.maximum(z + b1_ref[...], 0.0)
    o = lax.dot_general(w2_ref[...], z, (((1,), (1,)), ((), ())),
                        preferred_element_type=jnp.float32)
    o = o + b2_ref[...]
    out_ref[...] = o.reshape(o.shape[0], lb, B)


def _mlp_t(h, W1, b1, W2, b2, L, B, lb=8):
    n_tags = W2.shape[0]
    grid = (L // lb,)
    return pl.pallas_call(
        functools.partial(_mlp_body, lb, B),
        grid=grid,
        in_specs=[
            pl.BlockSpec((lb * B, D_EMB), lambda i: (i, 0)),
            pl.BlockSpec((D_EMB, D_EMB), lambda i: (0, 0)),
            pl.BlockSpec((1, D_EMB), lambda i: (0, 0)),
            pl.BlockSpec((n_tags, D_EMB), lambda i: (0, 0)),
            pl.BlockSpec((n_tags, 1), lambda i: (0, 0)),
        ],
        out_specs=pl.BlockSpec((n_tags, lb, B), lambda i: (0, i, 0)),
        out_shape=jax.ShapeDtypeStruct((n_tags, L, B), jnp.float32),
    )(h, W1, b1, W2, b2)


def kernel(x, emb, W1, b1, W2, b2):
    B, L = x.shape
    n_rows = B * L
    idx = x.T.reshape(NW, n_rows // NW).astype(jnp.int32)
    h = _make_gather(n_rows)(emb, idx)
    o_t = _mlp_t(h, W1, b1.reshape(1, -1), W2, b2.reshape(-1, 1), L, B)
    return jnp.transpose(o_t, (2, 1, 0))
